# baseline (speedup 1.0000x reference)
import functools

import jax
import jax.numpy as jnp
from jax import lax
from jax.experimental import pallas as pl
from jax.experimental.pallas import tpu as pltpu
from jax.experimental.pallas import tpu_sc as plsc

_L = 16
_CHUNK = 128


@functools.lru_cache(maxsize=None)
def _build(batch, embed_dim):
    info = plsc.get_sparse_core_info()
    nw = info.num_cores * info.num_subcores
    bpw = batch // nw
    nch = bpw // _CHUNK
    mesh = plsc.VectorSubcoreMesh(core_axis_name="c", subcore_axis_name="s")

    @functools.partial(
        pl.kernel,
        mesh=mesh,
        out_type=jax.ShapeDtypeStruct((batch,), jnp.float32),
        compiler_params=pltpu.CompilerParams(needs_layout_passes=False,
                                             use_tc_tiling_on_sc=False),
        scratch_types=[
            pltpu.VMEM((nch, _CHUNK), jnp.int32),
            pltpu.VMEM((nch, _CHUNK), jnp.int32),
            pltpu.VMEM((bpw, embed_dim), jnp.float32),
            pltpu.VMEM((bpw, embed_dim), jnp.float32),
            pltpu.VMEM((bpw,), jnp.float32),
            pltpu.VMEM((bpw,), jnp.float32),
            pltpu.VMEM((bpw,), jnp.float32),
            pltpu.VMEM((_L,), jnp.float32),
            pltpu.SemaphoreType.DMA,
        ],
    )
    def mf(src_hbm, dst_hbm, uemb_hbm, ubias_hbm, iemb_hbm, ibias_hbm,
           mean_hbm, out_hbm, sidx, didx, urows, irows, ub, ib, outv,
           meanv, sem):
        wid = lax.axis_index("s") * info.num_cores + lax.axis_index("c")
        base = wid * bpw

        pltpu.sync_copy(src_hbm.at[wid], sidx)
        pltpu.sync_copy(dst_hbm.at[wid], didx)
        pltpu.sync_copy(mean_hbm, meanv)

        descs = []
        for j in range(nch):
            rows = pl.ds(j * _CHUNK, _CHUNK)
            descs.append(pltpu.async_copy(uemb_hbm.at[sidx.at[j]],
                                          urows.at[rows, :], sem))
            descs.append(pltpu.async_copy(iemb_hbm.at[didx.at[j]],
                                          irows.at[rows, :], sem))
            descs.append(pltpu.async_copy(ubias_hbm.at[sidx.at[j]],
                                          ub.at[rows], sem))
            descs.append(pltpu.async_copy(ibias_hbm.at[didx.at[j]],
                                          ib.at[rows], sem))
        for d in descs:
            d.wait()

        mean_vec = meanv[...]
        lanes = lax.iota(jnp.int32, _L)

        def group_body(g, _):
            sl = pl.ds(g * _L, _L)
            out_vec = jnp.zeros((_L,), jnp.float32)
            for j in range(_L):
                r = g * _L + j
                acc = urows[r, pl.ds(0, _L)] * irows[r, pl.ds(0, _L)]
                for c in range(1, embed_dim // _L):
                    acc = acc + (urows[r, pl.ds(c * _L, _L)]
                                 * irows[r, pl.ds(c * _L, _L)])
                s = jnp.sum(acc)
                out_vec = jnp.where(lanes == j, s, out_vec)
            outv[sl] = out_vec + ub[sl] + ib[sl] + mean_vec
            return 0

        lax.fori_loop(0, bpw // _L, group_body, 0)

        pltpu.sync_copy(outv, out_hbm.at[pl.ds(base, bpw)])

    return mf, nw, nch


def kernel(src, dst, user_emb, user_bias, item_emb, item_bias, mean):
    batch = src.shape[0]
    embed_dim = user_emb.shape[1]
    mf, nw, nch = _build(batch, embed_dim)
    src3 = src.astype(jnp.int32).reshape(nw, nch, _CHUNK)
    dst3 = dst.astype(jnp.int32).reshape(nw, nch, _CHUNK)
    mean8 = jnp.broadcast_to(mean.reshape(()), (_L,)).astype(jnp.float32)
    return mf(src3, dst3, user_emb, user_bias.reshape(-1),
              item_emb, item_bias.reshape(-1), mean8)
